# Initial kernel scaffold; baseline (speedup 1.0000x reference)
import jax
import jax.numpy as jnp
from jax.experimental import pallas as pl


def kernel(x, edge_index, W, b):
    raise NotImplementedError("write your pallas kernel here")


# speedup vs baseline: 4.1413x; 4.1413x over previous
import functools

import jax
import jax.numpy as jnp
from jax import lax
from jax.experimental import pallas as pl
from jax.experimental.pallas import tpu as pltpu
from jax.experimental.pallas import tpu_sc as plsc

N = 10000
E = 160000
D = 128
H = 32

NC = 2
NS = 16
NW = NC * NS
CHUNK = 128
NCHUNKS = E // CHUNK
STRIPE = 624
TAIL = N - NS * STRIPE

_mesh = plsc.VectorSubcoreMesh(core_axis_name="c", subcore_axis_name="s")


def _worker_chunk_base(wid):
    nbase = NCHUNKS // NW
    extra = NCHUNKS % NW
    base = wid * nbase + jnp.minimum(wid, extra)
    count = nbase + jnp.where(wid < extra, 1, 0)
    return base, count


@functools.partial(
    pl.kernel,
    out_type=(
        jax.ShapeDtypeStruct((NC, 1, N), jnp.float32),
        jax.ShapeDtypeStruct((NC, 1, N), jnp.float32),
    ),
    mesh=_mesh,
    scratch_types=[
        pltpu.VMEM((CHUNK,), jnp.int32),
        pltpu.VMEM((CHUNK,), jnp.int32),
        pltpu.VMEM((CHUNK,), jnp.float32),
        pltpu.VMEM((N,), jnp.float32),
        pltpu.VMEM_SHARED((N,), jnp.float32),
        pltpu.VMEM_SHARED((N,), jnp.float32),
    ],
)
def _deg_call(src_hbm, dst_hbm, dego_hbm, degi_hbm, src_idx, dst_idx, ones_v,
              zeros_v, sh_dego, sh_degi):
    cid = lax.axis_index("c")
    sid = lax.axis_index("s")
    wid = cid * NS + sid

    for i in range(CHUNK // 16):
        ones_v[pl.ds(i * 16, 16)] = jnp.ones((16,), jnp.float32)

    @pl.when(sid == 0)
    def _init():
        def zbody(i, _):
            zeros_v[pl.ds(i * 16, 16)] = jnp.zeros((16,), jnp.float32)
            return ()
        lax.fori_loop(0, N // 16, zbody, ())
        pltpu.sync_copy(zeros_v, sh_dego)
        pltpu.sync_copy(zeros_v, sh_degi)

    plsc.subcore_barrier()

    base, count = _worker_chunk_base(wid)

    def body(c, _):
        chunk = base + c
        off = pl.multiple_of(chunk * CHUNK, CHUNK)
        pltpu.sync_copy(src_hbm.at[pl.ds(off, CHUNK)], src_idx)
        pltpu.sync_copy(dst_hbm.at[pl.ds(off, CHUNK)], dst_idx)
        pltpu.sync_copy(ones_v, sh_dego.at[src_idx], add=True)
        pltpu.sync_copy(ones_v, sh_degi.at[dst_idx], add=True)
        return ()

    lax.fori_loop(0, count, body, ())
    plsc.subcore_barrier()

    @pl.when(sid == 0)
    def _writeout():
        pltpu.sync_copy(sh_dego, dego_hbm.at[cid, 0])
        pltpu.sync_copy(sh_degi, degi_hbm.at[cid, 0])


@functools.partial(
    pl.kernel,
    out_type=jax.ShapeDtypeStruct((NC, N, D), jnp.float32),
    mesh=_mesh,
    scratch_types=[
        pltpu.VMEM((CHUNK,), jnp.int32),
        pltpu.VMEM((CHUNK,), jnp.int32),
        pltpu.VMEM((CHUNK, D), jnp.float32),
        pltpu.VMEM((48, D), jnp.float32),
        pltpu.VMEM_SHARED((N, D), jnp.float32),
        pltpu.SemaphoreType.DMA,
    ],
)
def _agg_call(src_hbm, dst_hbm, xs_hbm, agg_hbm, src_idx, dst_idx, rows_v,
              zrow_v, sh_agg, sem):
    cid = lax.axis_index("c")
    sid = lax.axis_index("s")
    wid = cid * NS + sid

    def zbody(i, _):
        for j in range(D // 16):
            zrow_v[i, pl.ds(j * 16, 16)] = jnp.zeros((16,), jnp.float32)
        return ()
    lax.fori_loop(0, 48, zbody, ())

    def zcopy(k, _):
        pltpu.sync_copy(zrow_v, sh_agg.at[pl.ds(sid * STRIPE + k * 48, 48)])
        return ()
    lax.fori_loop(0, STRIPE // 48, zcopy, ())

    @pl.when(sid == 0)
    def _init_tail():
        pltpu.sync_copy(zrow_v.at[pl.ds(0, TAIL)],
                        sh_agg.at[pl.ds(NS * STRIPE, TAIL)])

    plsc.subcore_barrier()

    base, count = _worker_chunk_base(wid)

    def body(c, _):
        chunk = base + c
        off = pl.multiple_of(chunk * CHUNK, CHUNK)
        pltpu.sync_copy(src_hbm.at[pl.ds(off, CHUNK)], src_idx)
        pltpu.sync_copy(dst_hbm.at[pl.ds(off, CHUNK)], dst_idx)
        pltpu.async_copy(xs_hbm.at[src_idx], rows_v, sem).wait()
        pltpu.sync_copy(rows_v, sh_agg.at[dst_idx], add=True)
        return ()

    lax.fori_loop(0, count, body, ())
    plsc.subcore_barrier()

    row0 = sid * STRIPE
    pltpu.sync_copy(sh_agg.at[pl.ds(row0, STRIPE)],
                    agg_hbm.at[cid, pl.ds(row0, STRIPE)])

    @pl.when(sid == 0)
    def _writeout_tail():
        pltpu.sync_copy(sh_agg.at[pl.ds(NS * STRIPE, TAIL)],
                        agg_hbm.at[cid, pl.ds(NS * STRIPE, TAIL)])


def _xs_body(x_ref, dego_ref, xs_ref):
    d = dego_ref[0, 0, :] + dego_ref[1, 0, :]
    norm = jnp.where(d > 0, lax.rsqrt(jnp.maximum(d, 1.0)), 0.0)
    xs_ref[...] = x_ref[...] * norm[:, None]


def _xs_call(x, dego):
    return pl.pallas_call(
        _xs_body,
        out_shape=jax.ShapeDtypeStruct((N, D), jnp.float32),
    )(x, dego)


def _z_body(agg_ref, w_ref, degi_ref, b_ref, z_ref):
    a = agg_ref[0] + agg_ref[1]
    d = degi_ref[0, 0, :] + degi_ref[1, 0, :]
    norm = jnp.where(d > 0, lax.rsqrt(jnp.maximum(d, 1.0)), 0.0)
    zw = jnp.dot(a, w_ref[...], preferred_element_type=jnp.float32)
    z_ref[...] = zw * norm[:, None] + b_ref[0, :][None, :]


def _z_call(aggp, W, degi, b2d):
    return pl.pallas_call(
        _z_body,
        out_shape=jax.ShapeDtypeStruct((N, H), jnp.float32),
    )(aggp, W, degi, b2d)


BI = 256
NBI = (N + BI - 1) // BI


def _recon_body(zi_ref, zall_ref, out_ref):
    out_ref[...] = lax.dot_general(
        zi_ref[...], zall_ref[...],
        dimension_numbers=(((1,), (1,)), ((), ())),
        preferred_element_type=jnp.float32,
    )


def _recon_call(z):
    return pl.pallas_call(
        _recon_body,
        grid=(NBI,),
        in_specs=[
            pl.BlockSpec((BI, H), lambda i: (i, 0)),
            pl.BlockSpec((N, H), lambda i: (0, 0)),
        ],
        out_specs=pl.BlockSpec((BI, N), lambda i: (i, 0)),
        out_shape=jax.ShapeDtypeStruct((N, N), jnp.float32),
    )(z, z)


def kernel(x, edge_index, W, b):
    src = edge_index[0]
    dst = edge_index[1]
    dego, degi = _deg_call(src, dst)
    xs = _xs_call(x, dego)
    aggp = _agg_call(src, dst, xs)
    z = _z_call(aggp, W, degi, b.reshape(1, H))
    recon = _recon_call(z)
    return (recon, z)
